# baseline (speedup 1.0000x reference)
import jax
import jax.numpy as jnp
import numpy as np
from jax import lax
from jax.experimental import pallas as pl
from jax.experimental.pallas import tpu as pltpu
from jax.experimental.pallas import tpu_sc as plsc

N = 100000
D = 128
SEG = 1000
SPAD = 1024
NC = 2
NS = 16
NW = NC * NS
RPW = N // NW
BLK = 125
NBLK = RPW // BLK
TPR = SPAD // NS
NBUF = 5
NPRE = 2
ALN = RPW + 11
L = 16


def _sc_body(x_hbm, idx_hbm, z2_hbm, z1_hbm, ones_hbm, psums_hbm, pcnts_hbm,
             xb, idx1d, idxv, onesv, acc, cacc, sems, semx, semc):
    c = lax.axis_index("c")
    s = lax.axis_index("s")
    wid = c * NS + s

    pltpu.sync_copy(z2_hbm.at[pl.ds(s * TPR, TPR)], acc.at[pl.ds(s * TPR, TPR)])
    pltpu.sync_copy(z1_hbm.at[pl.ds(s * TPR, TPR)], cacc.at[pl.ds(s * TPR, TPR)])
    a0 = pl.multiple_of(jnp.minimum((wid * RPW) & ~7, N - ALN), 8)
    off = wid * RPW - a0
    pltpu.sync_copy(idx_hbm.at[pl.ds(a0, ALN)], idx1d)
    pltpu.sync_copy(ones_hbm, onesv)
    iota = lax.iota(jnp.int32, L)
    offv = jnp.full((L,), off, dtype=jnp.int32)
    for j in range(NBLK):
        jv = jnp.full((L,), j, dtype=jnp.int32)
        for k in range(8):
            lanes = k * L + iota
            m = lanes < BLK
            vals = plsc.load_gather(idx1d, [offv + j * BLK + lanes], mask=m)
            plsc.store_scatter(idxv, [jv, lanes], vals, mask=m)
    plsc.subcore_barrier()

    loads = [None] * NBUF
    scats = [None] * NBUF
    dcs = [None, None]
    for j in range(NPRE):
        loads[j] = pltpu.async_copy(
            x_hbm.at[pl.ds(wid * RPW + j * BLK, BLK)], xb.at[j], sems[j])
    for j in range(NBLK):
        b = j % NBUF
        loads[b].wait()
        if dcs[j % 2] is not None:
            dcs[j % 2].wait()
        dcs[j % 2] = pltpu.async_copy(
            onesv.at[pl.ds(0, BLK)], cacc.at[idxv.at[j]], semc[j % 2], add=True)
        scats[b] = pltpu.async_copy(xb.at[b], acc.at[idxv.at[j]], semx[b],
                                    add=True)
        nj = j + NPRE
        if nj < NBLK:
            bn = nj % NBUF
            if scats[bn] is not None:
                scats[bn].wait()
            loads[bn] = pltpu.async_copy(
                x_hbm.at[pl.ds(wid * RPW + nj * BLK, BLK)], xb.at[bn], sems[bn])

    for d in scats + dcs:
        if d is not None:
            d.wait()
    plsc.subcore_barrier()
    pltpu.sync_copy(acc.at[pl.ds(s * TPR, TPR)], psums_hbm.at[c, pl.ds(s * TPR, TPR)])
    pltpu.sync_copy(cacc.at[pl.ds(s * TPR, TPR)], pcnts_hbm.at[c, pl.ds(s * TPR, TPR)])


_sc_pool = pl.kernel(
    _sc_body,
    out_type=(jax.ShapeDtypeStruct((NC, SPAD, D), jnp.float32),
              jax.ShapeDtypeStruct((NC, SPAD), jnp.float32)),
    mesh=plsc.VectorSubcoreMesh(core_axis_name="c", subcore_axis_name="s"),
    compiler_params=pltpu.CompilerParams(use_tc_tiling_on_sc=False,
                                         needs_layout_passes=False),
    scratch_types=[
        pltpu.VMEM((NBUF, BLK, D), jnp.float32),
        pltpu.VMEM((ALN,), jnp.int32),
        pltpu.VMEM((NBLK, BLK), jnp.int32),
        pltpu.VMEM((D,), jnp.float32),
        pltpu.VMEM_SHARED((SPAD, D), jnp.float32),
        pltpu.VMEM_SHARED((SPAD,), jnp.float32),
        [pltpu.SemaphoreType.DMA] * NBUF,
        [pltpu.SemaphoreType.DMA] * NBUF,
        [pltpu.SemaphoreType.DMA] * 2,
    ],
)


def _fin_body(scale_ref, ps_ref, pc_ref, o_ref):
    sums = ps_ref[0] + ps_ref[1]
    cnt = pc_ref[0] + pc_ref[1]
    cnt = jnp.maximum(cnt, 1.0).reshape(SPAD, 1)
    o_ref[...] = sums[:SEG] * (scale_ref[0, 0] / cnt[:SEG])


_Z2 = np.zeros((SPAD, D), np.float32)
_Z1 = np.zeros((SPAD,), np.float32)
_ONES = np.ones((D,), np.float32)


def kernel(x, batch_idx, num_graphs):
    psums, pcnts = _sc_pool(x, batch_idx, _Z2, _Z1, _ONES)
    scale = (jnp.asarray(num_graphs, jnp.float32) / jnp.float32(SEG)).reshape(1, 1)
    return pl.pallas_call(
        _fin_body,
        out_shape=jax.ShapeDtypeStruct((SEG, D), jnp.float32),
        in_specs=[
            pl.BlockSpec(memory_space=pltpu.SMEM),
            pl.BlockSpec(memory_space=pltpu.VMEM),
            pl.BlockSpec(memory_space=pltpu.VMEM),
        ],
        out_specs=pl.BlockSpec(memory_space=pltpu.VMEM),
    )(scale, psums, pcnts)


# speedup vs baseline: 1.0058x; 1.0058x over previous
import jax
import jax.numpy as jnp
import numpy as np
from jax import lax
from jax.experimental import pallas as pl
from jax.experimental.pallas import tpu as pltpu
from jax.experimental.pallas import tpu_sc as plsc

N = 100000
D = 128
SEG = 1000
SPAD = 1024
NC = 2
NS = 16
NW = NC * NS
RPW = N // NW
BLK = 125
NBLK = RPW // BLK
TPR = SPAD // NS
NBUF = 5
NPRE = 2


def _sc_body(x_hbm, idx_hbm, z2_hbm, z1_hbm, ones_hbm, psums_hbm, pcnts_hbm,
             xb, idxv, onesv, acc, cacc, sems, semx, semc):
    c = lax.axis_index("c")
    s = lax.axis_index("s")
    wid = c * NS + s

    pltpu.sync_copy(z2_hbm.at[pl.ds(s * TPR, TPR)], acc.at[pl.ds(s * TPR, TPR)])
    pltpu.sync_copy(z1_hbm.at[pl.ds(s * TPR, TPR)], cacc.at[pl.ds(s * TPR, TPR)])
    pltpu.sync_copy(idx_hbm.at[pl.ds(wid * NBLK, NBLK)], idxv)
    pltpu.sync_copy(ones_hbm, onesv)
    plsc.subcore_barrier()

    loads = [None] * NBUF
    scats = [None] * NBUF
    dcs = [None, None]
    for j in range(NPRE):
        loads[j] = pltpu.async_copy(
            x_hbm.at[pl.ds(wid * RPW + j * BLK, BLK)], xb.at[j], sems[j])
    for j in range(NBLK):
        b = j % NBUF
        loads[b].wait()
        if dcs[j % 2] is not None:
            dcs[j % 2].wait()
        dcs[j % 2] = pltpu.async_copy(
            onesv.at[pl.ds(0, BLK)], cacc.at[idxv.at[j]], semc[j % 2], add=True)
        scats[b] = pltpu.async_copy(xb.at[b], acc.at[idxv.at[j]], semx[b],
                                    add=True)
        nj = j + NPRE
        if nj < NBLK:
            bn = nj % NBUF
            if scats[bn] is not None:
                scats[bn].wait()
            loads[bn] = pltpu.async_copy(
                x_hbm.at[pl.ds(wid * RPW + nj * BLK, BLK)], xb.at[bn], sems[bn])

    for d in scats + dcs:
        if d is not None:
            d.wait()
    plsc.subcore_barrier()
    pltpu.sync_copy(acc.at[pl.ds(s * TPR, TPR)], psums_hbm.at[c, pl.ds(s * TPR, TPR)])
    pltpu.sync_copy(cacc.at[pl.ds(s * TPR, TPR)], pcnts_hbm.at[c, pl.ds(s * TPR, TPR)])


_sc_pool = pl.kernel(
    _sc_body,
    out_type=(jax.ShapeDtypeStruct((NC, SPAD, D), jnp.float32),
              jax.ShapeDtypeStruct((NC, SPAD), jnp.float32)),
    mesh=plsc.VectorSubcoreMesh(core_axis_name="c", subcore_axis_name="s"),
    compiler_params=pltpu.CompilerParams(use_tc_tiling_on_sc=False),
    scratch_types=[
        pltpu.VMEM((NBUF, BLK, D), jnp.float32),
        pltpu.VMEM((NBLK, BLK), jnp.int32),
        pltpu.VMEM((D,), jnp.float32),
        pltpu.VMEM_SHARED((SPAD, D), jnp.float32),
        pltpu.VMEM_SHARED((SPAD,), jnp.float32),
        [pltpu.SemaphoreType.DMA] * NBUF,
        [pltpu.SemaphoreType.DMA] * NBUF,
        [pltpu.SemaphoreType.DMA] * 2,
    ],
)


def _fin_body(scale_ref, ps_ref, pc_ref, o_ref):
    sums = ps_ref[0] + ps_ref[1]
    cnt = pc_ref[0] + pc_ref[1]
    cnt = jnp.maximum(cnt, 1.0).reshape(SPAD, 1)
    o_ref[...] = sums[:SEG] * (scale_ref[0, 0] / cnt[:SEG])


_Z2 = np.zeros((SPAD, D), np.float32)
_Z1 = np.zeros((SPAD,), np.float32)
_ONES = np.ones((D,), np.float32)


def kernel(x, batch_idx, num_graphs):
    idx2d = batch_idx.reshape(N // BLK, BLK)
    psums, pcnts = _sc_pool(x, idx2d, _Z2, _Z1, _ONES)
    scale = (jnp.asarray(num_graphs, jnp.float32) / jnp.float32(SEG)).reshape(1, 1)
    return pl.pallas_call(
        _fin_body,
        out_shape=jax.ShapeDtypeStruct((SEG, D), jnp.float32),
        in_specs=[
            pl.BlockSpec(memory_space=pltpu.SMEM),
            pl.BlockSpec(memory_space=pltpu.VMEM),
            pl.BlockSpec(memory_space=pltpu.VMEM),
        ],
        out_specs=pl.BlockSpec(memory_space=pltpu.VMEM),
    )(scale, psums, pcnts)


# speedup vs baseline: 1.0360x; 1.0300x over previous
import jax
import jax.numpy as jnp
import numpy as np
from jax import lax
from jax.experimental import pallas as pl
from jax.experimental.pallas import tpu as pltpu
from jax.experimental.pallas import tpu_sc as plsc

N = 100000
D = 128
SEG = 1000
SPAD = 1024
NC = 2
NS = 16
NW = NC * NS
RPW = N // NW
BLK = 125
NBLK = RPW // BLK
TPR = SPAD // NS
NBUF = 5
NPRE = 2


def _sc_body(x_hbm, idx_hbm, z2_hbm, z1_hbm, ones_hbm, psums_hbm, pcnts_hbm,
             xb, idxv, onesv, acc, cacc, sems, semx, semc):
    c = lax.axis_index("c")
    s = lax.axis_index("s")
    wid = c * NS + s

    pltpu.sync_copy(z2_hbm.at[pl.ds(s * TPR, TPR)], acc.at[pl.ds(s * TPR, TPR)])
    pltpu.sync_copy(z1_hbm.at[pl.ds(s * TPR, TPR)], cacc.at[pl.ds(s * TPR, TPR)])
    pltpu.sync_copy(idx_hbm.at[pl.ds(wid * NBLK, NBLK)], idxv)
    pltpu.sync_copy(ones_hbm, onesv)
    plsc.subcore_barrier()

    loads = [None] * NBUF
    scats = [None] * NBUF
    dcs = []
    for j in range(NPRE):
        loads[j] = pltpu.async_copy(
            x_hbm.at[pl.ds(wid * RPW + j * BLK, BLK)], xb.at[j], sems[j])
    for j in range(NBLK):
        b = j % NBUF
        loads[b].wait()
        dcs.append(pltpu.async_copy(
            onesv.at[pl.ds(0, BLK)], cacc.at[idxv.at[j]], semc[0], add=True))
        scats[b] = pltpu.async_copy(xb.at[b], acc.at[idxv.at[j]], semx[b],
                                    add=True)
        nj = j + NPRE
        if nj < NBLK:
            bn = nj % NBUF
            if scats[bn] is not None:
                scats[bn].wait()
            loads[bn] = pltpu.async_copy(
                x_hbm.at[pl.ds(wid * RPW + nj * BLK, BLK)], xb.at[bn], sems[bn])

    for d in scats + dcs:
        if d is not None:
            d.wait()
    plsc.subcore_barrier()
    pltpu.sync_copy(acc.at[pl.ds(s * TPR, TPR)], psums_hbm.at[c, pl.ds(s * TPR, TPR)])
    pltpu.sync_copy(cacc.at[pl.ds(s * TPR, TPR)], pcnts_hbm.at[c, pl.ds(s * TPR, TPR)])


_sc_pool = pl.kernel(
    _sc_body,
    out_type=(jax.ShapeDtypeStruct((NC, SPAD, D), jnp.float32),
              jax.ShapeDtypeStruct((NC, SPAD), jnp.float32)),
    mesh=plsc.VectorSubcoreMesh(core_axis_name="c", subcore_axis_name="s"),
    compiler_params=pltpu.CompilerParams(use_tc_tiling_on_sc=False),
    scratch_types=[
        pltpu.VMEM((NBUF, BLK, D), jnp.float32),
        pltpu.VMEM((NBLK, BLK), jnp.int32),
        pltpu.VMEM((D,), jnp.float32),
        pltpu.VMEM_SHARED((SPAD, D), jnp.float32),
        pltpu.VMEM_SHARED((SPAD,), jnp.float32),
        [pltpu.SemaphoreType.DMA] * NBUF,
        [pltpu.SemaphoreType.DMA] * NBUF,
        [pltpu.SemaphoreType.DMA] * 2,
    ],
)


def _fin_body(scale_ref, ps_ref, pc_ref, o_ref):
    sums = ps_ref[0] + ps_ref[1]
    cnt = pc_ref[0] + pc_ref[1]
    cnt = jnp.maximum(cnt, 1.0).reshape(SPAD, 1)
    o_ref[...] = sums[:SEG] * (scale_ref[0, 0] / cnt[:SEG])


_Z2 = np.zeros((SPAD, D), np.float32)
_Z1 = np.zeros((SPAD,), np.float32)
_ONES = np.ones((D,), np.float32)


def kernel(x, batch_idx, num_graphs):
    idx2d = batch_idx.reshape(N // BLK, BLK)
    psums, pcnts = _sc_pool(x, idx2d, _Z2, _Z1, _ONES)
    scale = (jnp.asarray(num_graphs, jnp.float32) / jnp.float32(SEG)).reshape(1, 1)
    return pl.pallas_call(
        _fin_body,
        out_shape=jax.ShapeDtypeStruct((SEG, D), jnp.float32),
        in_specs=[
            pl.BlockSpec(memory_space=pltpu.SMEM),
            pl.BlockSpec(memory_space=pltpu.VMEM),
            pl.BlockSpec(memory_space=pltpu.VMEM),
        ],
        out_specs=pl.BlockSpec(memory_space=pltpu.VMEM),
    )(scale, psums, pcnts)
